# baseline (speedup 1.0000x reference)
import functools

import jax
import jax.numpy as jnp
from jax import lax
from jax.experimental import pallas as pl
from jax.experimental.pallas import tpu as pltpu
from jax.experimental.pallas import tpu_sc as plsc

BATCH = 16384
DIM = 64
NC = 2
NS = 16
NW = NC * NS
BPW = BATCH // NW
CHUNK = 128
CHUNKS = BPW // CHUNK


def _sc_gather3(user_table, item_table, uidx, pidx, nidx):
    mesh = plsc.VectorSubcoreMesh(core_axis_name="c", subcore_axis_name="s")
    out = jax.ShapeDtypeStruct((BATCH, DIM), jnp.float32)

    @functools.partial(
        pl.kernel,
        out_type=(out, out, out),
        mesh=mesh,
        compiler_params=pltpu.CompilerParams(use_tc_tiling_on_sc=False),
        scratch_types=[
            pltpu.VMEM((CHUNKS, CHUNK), jnp.int32),
            pltpu.VMEM((CHUNKS, CHUNK), jnp.int32),
            pltpu.VMEM((CHUNKS, CHUNK), jnp.int32),
            pltpu.VMEM((BPW, DIM), jnp.float32),
            pltpu.VMEM((BPW, DIM), jnp.float32),
            pltpu.VMEM((BPW, DIM), jnp.float32),
            pltpu.SemaphoreType.DMA,
        ],
    )
    def k(user_hbm, item_hbm, ui_hbm, pi_hbm, ni_hbm,
          out_u, out_p, out_n,
          idx_u, idx_p, idx_n, rows_u, rows_p, rows_n, sem):
        wid = lax.axis_index("s") * NC + lax.axis_index("c")
        irow = wid * CHUNKS
        base = wid * BPW
        pltpu.sync_copy(ui_hbm.at[pl.ds(irow, CHUNKS)], idx_u)
        pltpu.sync_copy(pi_hbm.at[pl.ds(irow, CHUNKS)], idx_p)
        pltpu.sync_copy(ni_hbm.at[pl.ds(irow, CHUNKS)], idx_n)
        cps = []
        for c in range(CHUNKS):
            dst = pl.ds(c * CHUNK, CHUNK)
            cps.append(pltpu.async_copy(user_hbm.at[idx_u.at[c]], rows_u.at[dst], sem))
            cps.append(pltpu.async_copy(item_hbm.at[idx_p.at[c]], rows_p.at[dst], sem))
            cps.append(pltpu.async_copy(item_hbm.at[idx_n.at[c]], rows_n.at[dst], sem))
        for cp in cps:
            cp.wait()
        pltpu.sync_copy(rows_u, out_u.at[pl.ds(base, BPW)])
        pltpu.sync_copy(rows_p, out_p.at[pl.ds(base, BPW)])
        pltpu.sync_copy(rows_n, out_n.at[pl.ds(base, BPW)])

    return k(user_table, item_table, uidx, pidx, nidx)


@jax.jit
def kernel(user, pos, neg, user_table, item_table):
    uidx = jnp.asarray(user, jnp.int32).reshape(BATCH // CHUNK, CHUNK)
    pidx = jnp.asarray(pos, jnp.int32).reshape(BATCH // CHUNK, CHUNK)
    nidx = jnp.asarray(neg, jnp.int32).reshape(BATCH // CHUNK, CHUNK)
    return _sc_gather3(user_table, item_table, uidx, pidx, nidx)


# speedup vs baseline: 1.6278x; 1.6278x over previous
import functools

import jax
import jax.numpy as jnp
from jax import lax
from jax.experimental import pallas as pl
from jax.experimental.pallas import tpu as pltpu
from jax.experimental.pallas import tpu_sc as plsc

BATCH = 16384
DIM = 64
NC = 2
NS = 16
NW = NC * NS
BPW = BATCH // NW
LANES = 16


def _sc_gather3(user_table, item_table, uidx, pidx, nidx):
    mesh = plsc.VectorSubcoreMesh(core_axis_name="c", subcore_axis_name="s")
    out = jax.ShapeDtypeStruct((BATCH, DIM), jnp.float32)

    @functools.partial(
        pl.kernel,
        out_type=(out, out, out),
        mesh=mesh,
        scratch_types=[
            pltpu.VMEM((BPW,), jnp.int32),
            pltpu.VMEM((BPW, DIM), jnp.float32),
            pltpu.SemaphoreType.DMA,
        ],
    )
    def k(user_hbm, item_hbm, ui_hbm, pi_hbm, ni_hbm,
          out_u, out_p, out_n,
          idxbuf, gbuf, gsem):
        wid = lax.axis_index("s") * NC + lax.axis_index("c")
        base = wid * BPW

        def do_table(tbl, idx_hbm, out_hbm):
            pltpu.sync_copy(idx_hbm.at[pl.ds(base, BPW)], idxbuf)

            def grp(g, carry):
                gb = g * LANES
                v = idxbuf[pl.ds(gb, LANES)]
                for j in range(LANES):
                    pltpu.async_copy(
                        tbl.at[pl.ds(v[j], 1)], gbuf.at[pl.ds(gb + j, 1)], gsem)
                return carry

            lax.fori_loop(0, BPW // LANES, grp, 0)
            pltpu.make_async_copy(tbl.at[pl.ds(0, BPW)], gbuf, gsem).wait()
            pltpu.sync_copy(gbuf, out_hbm.at[pl.ds(base, BPW)])

        do_table(user_hbm, ui_hbm, out_u)
        do_table(item_hbm, pi_hbm, out_p)
        do_table(item_hbm, ni_hbm, out_n)

    return k(user_table, item_table, uidx, pidx, nidx)


@jax.jit
def kernel(user, pos, neg, user_table, item_table):
    uidx = jnp.asarray(user, jnp.int32)
    pidx = jnp.asarray(pos, jnp.int32)
    nidx = jnp.asarray(neg, jnp.int32)
    return _sc_gather3(user_table, item_table, uidx, pidx, nidx)
